# baseline (device time: 62904 ns/iter reference)
import jax
import jax.numpy as jnp
from jax import lax
from jax.experimental import pallas as pl
from jax.experimental.pallas import tpu as pltpu

N_DEV = 4
SQ = 1024
SKV = 1024
H_LOC = 8
DH = 128
D_MODEL = 1024
D_QKV = H_LOC * DH
SCALE = 0.08838834764831843
BLK = 64
CHUNK = 256
SUB = 128
N_SUB = SQ // SUB


def kernel(x, Wq, K_ext, V_ext, Wo):
    my_pos = lax.axis_index("i")

    x2 = x[0].astype(jnp.bfloat16)
    Wq_loc = (lax.dynamic_slice(
        Wq, (0, my_pos * D_QKV), (D_MODEL, D_QKV)) * SCALE).astype(jnp.bfloat16)
    Wo_loc = lax.dynamic_slice(
        Wo, (my_pos * D_QKV, 0), (D_QKV, D_MODEL)).astype(jnp.bfloat16)
    K = jnp.transpose(K_ext[0], (1, 0, 2)).astype(jnp.bfloat16)
    V = jnp.transpose(V_ext[0], (1, 0, 2)).astype(jnp.bfloat16)

    def body(x_ref, wq_ref, k_ref, v_ref, wo_ref, out_ref,
             stage_ref, rs_ref, ag_ref,
             rs_send_sems, rs_recv_sems, ag_send_sems, ag_recv_sems):
        my = lax.axis_index("i")

        barrier_sem = pltpu.get_barrier_semaphore()
        for d in range(1, N_DEV):
            pl.semaphore_signal(
                barrier_sem, inc=1,
                device_id=(lax.rem(my + d, N_DEV),),
                device_id_type=pl.DeviceIdType.MESH,
            )
        pl.semaphore_wait(barrier_sem, N_DEV - 1)

        def rs_send_desc(j):
            return pltpu.make_async_remote_copy(
                src_ref=stage_ref.at[j],
                dst_ref=rs_ref.at[my * 2 + j // N_DEV],
                send_sem=rs_send_sems.at[j],
                recv_sem=rs_recv_sems.at[my * 2 + j // N_DEV],
                device_id=(j % N_DEV,),
                device_id_type=pl.DeviceIdType.MESH,
            )

        def rs_recv_desc(s, half):
            return pltpu.make_async_remote_copy(
                src_ref=stage_ref.at[s],
                dst_ref=rs_ref.at[s * 2 + half],
                send_sem=rs_send_sems.at[s],
                recv_sem=rs_recv_sems.at[s * 2 + half],
                device_id=(s,),
                device_id_type=pl.DeviceIdType.MESH,
            )

        def ag_send_desc(j, d):
            return pltpu.make_async_remote_copy(
                src_ref=ag_ref.at[j],
                dst_ref=ag_ref.at[j],
                send_sem=ag_send_sems.at[(j // N_DEV) * N_DEV + d],
                recv_sem=ag_recv_sems.at[j],
                device_id=(d,),
                device_id_type=pl.DeviceIdType.MESH,
            )

        def reduce_and_broadcast(j):
            own = j % N_DEV
            half = j // N_DEV
            for s_id in range(N_DEV):
                if s_id != own:
                    rs_recv_desc(s_id, half).wait_recv()
            red = (rs_ref[0 * 2 + half].astype(jnp.float32)
                   + rs_ref[1 * 2 + half].astype(jnp.float32)
                   + rs_ref[2 * 2 + half].astype(jnp.float32)
                   + rs_ref[3 * 2 + half].astype(jnp.float32))
            ag_ref[j] = red.astype(jnp.bfloat16)
            for d in range(N_DEV):
                if d != own:
                    ag_send_desc(j, d).start()

        rb = lax.broadcasted_iota(jnp.int32, (CHUNK, CHUNK), 0) // BLK
        cb = lax.broadcasted_iota(jnp.int32, (CHUNK, CHUNK), 1) // BLK
        diag_bias = jnp.where(cb <= rb, 0.0, -1e9).astype(jnp.float32)

        for c in range(N_DEV):
            xc = x_ref[pl.ds(c * CHUNK, CHUNK), :]
            qc = jax.lax.dot(xc, wq_ref[...],
                             preferred_element_type=jnp.float32
                             ).astype(jnp.bfloat16)

            ctx_cols = []
            for h in range(H_LOC):
                qh = qc[:, h * DH:(h + 1) * DH]
                vh = v_ref[h, pl.ds(0, (c + 1) * CHUNK), :]
                kd = k_ref[h, pl.ds(c * CHUNK, CHUNK), :]
                sd = lax.dot_general(
                    qh, kd, (((1,), (1,)), ((), ())),
                    preferred_element_type=jnp.float32)
                wd = jnp.exp(sd + diag_bias)
                if c > 0:
                    kf = k_ref[h, pl.ds(0, c * CHUNK), :]
                    sf = lax.dot_general(
                        qh, kf, (((1,), (1,)), ((), ())),
                        preferred_element_type=jnp.float32)
                    w = jnp.concatenate([jnp.exp(sf), wd], axis=1)
                else:
                    w = wd
                denom = jnp.sum(w, axis=-1, keepdims=True)
                ctx_raw = jax.lax.dot(
                    w.astype(jnp.bfloat16), vh,
                    preferred_element_type=jnp.float32)
                ctx_cols.append(ctx_raw * (1.0 / denom))
            ctx = jnp.concatenate(ctx_cols, axis=1).astype(jnp.bfloat16)
            pc = jax.lax.dot(ctx, wo_ref[...],
                             preferred_element_type=jnp.float32)
            pcb = pc.astype(jnp.bfloat16)

            for sub in range(2):
                j = 2 * c + sub
                half_val = pcb[sub * SUB:(sub + 1) * SUB, :]

                @pl.when(j % N_DEV == my)
                def _():
                    rs_ref[(j % N_DEV) * 2 + j // N_DEV] = half_val

                @pl.when(j % N_DEV != my)
                def _():
                    stage_ref[j] = half_val
                    rs_send_desc(j).start()

            if c >= 1:
                for j in (2 * (c - 1), 2 * (c - 1) + 1):
                    @pl.when(j % N_DEV == my)
                    def _():
                        reduce_and_broadcast(j)

        for j in (N_SUB - 2, N_SUB - 1):
            @pl.when(j % N_DEV == my)
            def _():
                reduce_and_broadcast(j)

        for j in range(N_SUB):
            @pl.when(j % N_DEV != my)
            def _():
                pltpu.make_async_remote_copy(
                    src_ref=ag_ref.at[j],
                    dst_ref=ag_ref.at[j],
                    send_sem=ag_send_sems.at[j],
                    recv_sem=ag_recv_sems.at[j],
                    device_id=(j % N_DEV,),
                    device_id_type=pl.DeviceIdType.MESH,
                ).wait_recv()
                out_ref[0, pl.ds(j * SUB, SUB), :] = (
                    ag_ref[j].astype(jnp.float32))

            @pl.when(j % N_DEV == my)
            def _():
                out_ref[0, pl.ds(j * SUB, SUB), :] = (
                    ag_ref[j].astype(jnp.float32))

        for j in range(N_SUB):
            @pl.when(j % N_DEV != my)
            def _():
                rs_send_desc(j).wait_send()
        for half in range(2):
            for d in range(N_DEV):
                @pl.when(d != my)
                def _():
                    pltpu.make_async_remote_copy(
                        src_ref=ag_ref.at[half],
                        dst_ref=ag_ref.at[half],
                        send_sem=ag_send_sems.at[half * N_DEV + d],
                        recv_sem=ag_recv_sems.at[half],
                        device_id=(d,),
                        device_id_type=pl.DeviceIdType.MESH,
                    ).wait_send()

    return pl.pallas_call(
        body,
        out_shape=jax.ShapeDtypeStruct((1, SQ, D_MODEL), jnp.float32),
        in_specs=[pl.BlockSpec(memory_space=pltpu.VMEM)] * 5,
        out_specs=pl.BlockSpec(memory_space=pltpu.VMEM),
        scratch_shapes=[
            pltpu.VMEM((N_SUB, SUB, D_MODEL), jnp.bfloat16),
            pltpu.VMEM((N_DEV * 2, SUB, D_MODEL), jnp.bfloat16),
            pltpu.VMEM((N_SUB, SUB, D_MODEL), jnp.bfloat16),
            pltpu.SemaphoreType.DMA((N_SUB,)),
            pltpu.SemaphoreType.DMA((N_DEV * 2,)),
            pltpu.SemaphoreType.DMA((N_SUB,)),
            pltpu.SemaphoreType.DMA((N_SUB,)),
        ],
        compiler_params=pltpu.CompilerParams(
            collective_id=0, vmem_limit_bytes=100 * 1024 * 1024),
    )(x2, Wq_loc, K, V, Wo_loc)


# device time: 48675 ns/iter; 1.2923x vs baseline; 1.2923x over previous
import jax
import jax.numpy as jnp
from jax import lax
from jax.experimental import pallas as pl
from jax.experimental.pallas import tpu as pltpu

N_DEV = 4
SQ = 1024
SKV = 1024
H_LOC = 8
DH = 128
D_MODEL = 1024
D_QKV = H_LOC * DH
SCALE = 0.08838834764831843
BLK = 64
CHUNK = 256
SUB = 128
N_SUB = SQ // SUB


def kernel(x, Wq, K_ext, V_ext, Wo):
    def body(x_hbm, wq_hbm, k_hbm, v_hbm, wo_hbm, out_ref,
             x_v, wq_v, kT_v, vT_v, wo_v,
             wqb, wob, kT, vT,
             stage_ref, rs_ref, ag_ref,
             load_sems, tr_sems,
             rs_send_sems, rs_recv_sems, ag_send_sems, ag_recv_sems):
        my = lax.axis_index("i")

        tr_copies = []
        for h in range(H_LOC):
            tr_copies.append(pltpu.make_async_copy(
                k_hbm.at[0, :, h, :], kT_v.at[h], tr_sems.at[h]))
            tr_copies.append(pltpu.make_async_copy(
                v_hbm.at[0, :, h, :], vT_v.at[h], tr_sems.at[H_LOC + h]))
        for cp in tr_copies:
            cp.start()
        cp_x = pltpu.make_async_copy(x_hbm.at[0], x_v, load_sems.at[0])
        cp_wq = pltpu.make_async_copy(
            wq_hbm.at[:, pl.ds(my * D_QKV, D_QKV)], wq_v, load_sems.at[1])
        cp_wo = pltpu.make_async_copy(
            wo_hbm.at[pl.ds(my * D_QKV, D_QKV), :], wo_v, load_sems.at[4])
        for cp in (cp_x, cp_wq, cp_wo):
            cp.start()

        barrier_sem = pltpu.get_barrier_semaphore()
        for d in range(1, N_DEV):
            pl.semaphore_signal(
                barrier_sem, inc=1,
                device_id=(lax.rem(my + d, N_DEV),),
                device_id_type=pl.DeviceIdType.MESH,
            )
        pl.semaphore_wait(barrier_sem, N_DEV - 1)

        cp_wq.wait()
        wqb[...] = (wq_v[...] * SCALE).astype(jnp.bfloat16)
        cp_wo.wait()
        wob[...] = wo_v[...].astype(jnp.bfloat16)
        for cp in tr_copies:
            cp.wait()
        kT[...] = kT_v[...].astype(jnp.bfloat16)
        vT[...] = vT_v[...].astype(jnp.bfloat16)
        cp_x.wait()

        def rs_send_desc(j):
            return pltpu.make_async_remote_copy(
                src_ref=stage_ref.at[j],
                dst_ref=rs_ref.at[my * 2 + j // N_DEV],
                send_sem=rs_send_sems.at[j],
                recv_sem=rs_recv_sems.at[my * 2 + j // N_DEV],
                device_id=(j % N_DEV,),
                device_id_type=pl.DeviceIdType.MESH,
            )

        def rs_recv_desc(s, half):
            return pltpu.make_async_remote_copy(
                src_ref=stage_ref.at[s],
                dst_ref=rs_ref.at[s * 2 + half],
                send_sem=rs_send_sems.at[s],
                recv_sem=rs_recv_sems.at[s * 2 + half],
                device_id=(s,),
                device_id_type=pl.DeviceIdType.MESH,
            )

        def ag_send_desc(j, d):
            return pltpu.make_async_remote_copy(
                src_ref=ag_ref.at[j],
                dst_ref=ag_ref.at[j],
                send_sem=ag_send_sems.at[(j // N_DEV) * N_DEV + d],
                recv_sem=ag_recv_sems.at[j],
                device_id=(d,),
                device_id_type=pl.DeviceIdType.MESH,
            )

        def reduce_and_broadcast(j):
            own = j % N_DEV
            half = j // N_DEV
            for s_id in range(N_DEV):
                if s_id != own:
                    rs_recv_desc(s_id, half).wait_recv()
            red = (rs_ref[0 * 2 + half].astype(jnp.float32)
                   + rs_ref[1 * 2 + half].astype(jnp.float32)
                   + rs_ref[2 * 2 + half].astype(jnp.float32)
                   + rs_ref[3 * 2 + half].astype(jnp.float32))
            ag_ref[j] = red.astype(jnp.bfloat16)
            for d in range(N_DEV):
                if d != own:
                    ag_send_desc(j, d).start()

        rb = lax.broadcasted_iota(jnp.int32, (CHUNK, CHUNK), 0) // BLK
        cb = lax.broadcasted_iota(jnp.int32, (CHUNK, CHUNK), 1) // BLK
        diag_bias = jnp.where(cb <= rb, 0.0, -1e9).astype(jnp.float32)

        for c in range(N_DEV):
            xc = x_v[pl.ds(c * CHUNK, CHUNK), :].astype(jnp.bfloat16)
            qc = jax.lax.dot(xc, wqb[...],
                             preferred_element_type=jnp.float32
                             ).astype(jnp.bfloat16)

            ctx_cols = []
            for h in range(H_LOC):
                qh = qc[:, h * DH:(h + 1) * DH]
                vh = vT[h, pl.ds(0, (c + 1) * CHUNK), :]
                kd = kT[h, pl.ds(c * CHUNK, CHUNK), :]
                sd = lax.dot_general(
                    qh, kd, (((1,), (1,)), ((), ())),
                    preferred_element_type=jnp.float32)
                wd = jnp.exp(sd + diag_bias)
                if c > 0:
                    kf = kT[h, pl.ds(0, c * CHUNK), :]
                    sf = lax.dot_general(
                        qh, kf, (((1,), (1,)), ((), ())),
                        preferred_element_type=jnp.float32)
                    w = jnp.concatenate([jnp.exp(sf), wd], axis=1)
                else:
                    w = wd
                denom = jnp.sum(w, axis=-1, keepdims=True)
                ctx_raw = jax.lax.dot(
                    w.astype(jnp.bfloat16), vh,
                    preferred_element_type=jnp.float32)
                ctx_cols.append(ctx_raw * (1.0 / denom))
            ctx = jnp.concatenate(ctx_cols, axis=1).astype(jnp.bfloat16)
            pc = jax.lax.dot(ctx, wob[...],
                             preferred_element_type=jnp.float32)
            pcb = pc.astype(jnp.bfloat16)

            for sub in range(2):
                j = 2 * c + sub
                half_val = pcb[sub * SUB:(sub + 1) * SUB, :]

                @pl.when(j % N_DEV == my)
                def _():
                    rs_ref[(j % N_DEV) * 2 + j // N_DEV] = half_val

                @pl.when(j % N_DEV != my)
                def _():
                    stage_ref[j] = half_val
                    rs_send_desc(j).start()

            if c >= 1:
                for j in (2 * (c - 1), 2 * (c - 1) + 1):
                    @pl.when(j % N_DEV == my)
                    def _():
                        reduce_and_broadcast(j)

        for j in (N_SUB - 2, N_SUB - 1):
            @pl.when(j % N_DEV == my)
            def _():
                reduce_and_broadcast(j)

        for j in range(N_SUB):
            @pl.when(j % N_DEV != my)
            def _():
                pltpu.make_async_remote_copy(
                    src_ref=ag_ref.at[j],
                    dst_ref=ag_ref.at[j],
                    send_sem=ag_send_sems.at[j],
                    recv_sem=ag_recv_sems.at[j],
                    device_id=(j % N_DEV,),
                    device_id_type=pl.DeviceIdType.MESH,
                ).wait_recv()
                out_ref[0, pl.ds(j * SUB, SUB), :] = (
                    ag_ref[j].astype(jnp.float32))

            @pl.when(j % N_DEV == my)
            def _():
                out_ref[0, pl.ds(j * SUB, SUB), :] = (
                    ag_ref[j].astype(jnp.float32))

        for j in range(N_SUB):
            @pl.when(j % N_DEV != my)
            def _():
                rs_send_desc(j).wait_send()
        for half in range(2):
            for d in range(N_DEV):
                @pl.when(d != my)
                def _():
                    pltpu.make_async_remote_copy(
                        src_ref=ag_ref.at[half],
                        dst_ref=ag_ref.at[half],
                        send_sem=ag_send_sems.at[half * N_DEV + d],
                        recv_sem=ag_recv_sems.at[half],
                        device_id=(d,),
                        device_id_type=pl.DeviceIdType.MESH,
                    ).wait_send()

    return pl.pallas_call(
        body,
        out_shape=jax.ShapeDtypeStruct((1, SQ, D_MODEL), jnp.float32),
        in_specs=[pl.BlockSpec(memory_space=pl.ANY)] * 5,
        out_specs=pl.BlockSpec(memory_space=pltpu.VMEM),
        scratch_shapes=[
            pltpu.VMEM((SQ, D_MODEL), jnp.float32),
            pltpu.VMEM((D_MODEL, D_QKV), jnp.float32),
            pltpu.VMEM((H_LOC, SKV, DH), jnp.float32),
            pltpu.VMEM((H_LOC, SKV, DH), jnp.float32),
            pltpu.VMEM((D_QKV, D_MODEL), jnp.float32),
            pltpu.VMEM((D_MODEL, D_QKV), jnp.bfloat16),
            pltpu.VMEM((D_QKV, D_MODEL), jnp.bfloat16),
            pltpu.VMEM((H_LOC, SKV, DH), jnp.bfloat16),
            pltpu.VMEM((H_LOC, SKV, DH), jnp.bfloat16),
            pltpu.VMEM((N_SUB, SUB, D_MODEL), jnp.bfloat16),
            pltpu.VMEM((N_DEV * 2, SUB, D_MODEL), jnp.bfloat16),
            pltpu.VMEM((N_SUB, SUB, D_MODEL), jnp.bfloat16),
            pltpu.SemaphoreType.DMA((5,)),
            pltpu.SemaphoreType.DMA((2 * H_LOC,)),
            pltpu.SemaphoreType.DMA((N_SUB,)),
            pltpu.SemaphoreType.DMA((N_DEV * 2,)),
            pltpu.SemaphoreType.DMA((N_SUB,)),
            pltpu.SemaphoreType.DMA((N_SUB,)),
        ],
        compiler_params=pltpu.CompilerParams(
            collective_id=0, vmem_limit_bytes=110 * 1024 * 1024),
    )(x, Wq, K_ext, V_ext, Wo)


# device time: 45948 ns/iter; 1.3690x vs baseline; 1.0593x over previous
import jax
import jax.numpy as jnp
from jax import lax
from jax.experimental import pallas as pl
from jax.experimental.pallas import tpu as pltpu

N_DEV = 4
SQ = 1024
SKV = 1024
H_LOC = 8
DH = 128
D_MODEL = 1024
D_QKV = H_LOC * DH
SCALE = 0.08838834764831843
BLK = 64
CHUNK = 256
SUB = 64
N_SUB = SQ // SUB
SPC = CHUNK // SUB


def kernel(x, Wq, K_ext, V_ext, Wo):
    def body(x_hbm, wq_hbm, k_hbm, v_hbm, wo_hbm, out_ref,
             x_v, wq_v, kT_v, vT_v, wo_v,
             wqb, wob, kT, vT,
             stage_ref, rs_ref, ag_ref,
             load_sems, tr_sems,
             rs_send_sems, rs_recv_sems, ag_send_sems, ag_recv_sems):
        my = lax.axis_index("i")

        tr_k = [pltpu.make_async_copy(
            k_hbm.at[0, :, h, :], kT_v.at[h], tr_sems.at[h])
            for h in range(H_LOC)]
        tr_v = [pltpu.make_async_copy(
            v_hbm.at[0, :, h, :], vT_v.at[h], tr_sems.at[H_LOC + h])
            for h in range(H_LOC)]
        for h in range(H_LOC):
            tr_k[h].start()
            tr_v[h].start()
        cp_x = pltpu.make_async_copy(x_hbm.at[0], x_v, load_sems.at[0])
        cp_wq = pltpu.make_async_copy(
            wq_hbm.at[:, pl.ds(my * D_QKV, D_QKV)], wq_v, load_sems.at[1])
        cp_wo = pltpu.make_async_copy(
            wo_hbm.at[pl.ds(my * D_QKV, D_QKV), :], wo_v, load_sems.at[2])
        for cp in (cp_x, cp_wq, cp_wo):
            cp.start()

        barrier_sem = pltpu.get_barrier_semaphore()
        for d in range(1, N_DEV):
            pl.semaphore_signal(
                barrier_sem, inc=1,
                device_id=(lax.rem(my + d, N_DEV),),
                device_id_type=pl.DeviceIdType.MESH,
            )
        pl.semaphore_wait(barrier_sem, N_DEV - 1)

        cp_wq.wait()
        wqb[...] = (wq_v[...] * SCALE).astype(jnp.bfloat16)
        cp_wo.wait()
        wob[...] = wo_v[...].astype(jnp.bfloat16)
        cp_x.wait()

        def rs_send_desc(j):
            return pltpu.make_async_remote_copy(
                src_ref=stage_ref.at[j],
                dst_ref=rs_ref.at[my * SPC + j // N_DEV],
                send_sem=rs_send_sems.at[j],
                recv_sem=rs_recv_sems.at[my * SPC + j // N_DEV],
                device_id=(j % N_DEV,),
                device_id_type=pl.DeviceIdType.MESH,
            )

        def rs_recv_desc(s, q):
            return pltpu.make_async_remote_copy(
                src_ref=stage_ref.at[s],
                dst_ref=rs_ref.at[s * SPC + q],
                send_sem=rs_send_sems.at[s],
                recv_sem=rs_recv_sems.at[s * SPC + q],
                device_id=(s,),
                device_id_type=pl.DeviceIdType.MESH,
            )

        def ag_send_desc(j, d):
            return pltpu.make_async_remote_copy(
                src_ref=ag_ref.at[j],
                dst_ref=ag_ref.at[j],
                send_sem=ag_send_sems.at[(j // N_DEV) * N_DEV + d],
                recv_sem=ag_recv_sems.at[j],
                device_id=(d,),
                device_id_type=pl.DeviceIdType.MESH,
            )

        def reduce_and_broadcast(j):
            own = j % N_DEV
            q = j // N_DEV
            for s_id in range(N_DEV):
                if s_id != own:
                    rs_recv_desc(s_id, q).wait_recv()
            red = (rs_ref[0 * SPC + q].astype(jnp.float32)
                   + rs_ref[1 * SPC + q].astype(jnp.float32)
                   + rs_ref[2 * SPC + q].astype(jnp.float32)
                   + rs_ref[3 * SPC + q].astype(jnp.float32))
            ag_ref[j] = red.astype(jnp.bfloat16)
            for d in range(N_DEV):
                if d != own:
                    ag_send_desc(j, d).start()

        rb = lax.broadcasted_iota(jnp.int32, (CHUNK, CHUNK), 0) // BLK
        cb = lax.broadcasted_iota(jnp.int32, (CHUNK, CHUNK), 1) // BLK
        diag_bias = jnp.where(cb <= rb, 0.0, -1e9).astype(jnp.float32)

        for c in range(N_DEV):
            xc = x_v[pl.ds(c * CHUNK, CHUNK), :].astype(jnp.bfloat16)
            qc = jax.lax.dot(xc, wqb[...],
                             preferred_element_type=jnp.float32
                             ).astype(jnp.bfloat16)

            ctx_cols = []
            for h in range(H_LOC):
                if c == 0:
                    tr_k[h].wait()
                    kT[h] = kT_v[h].astype(jnp.bfloat16)
                    tr_v[h].wait()
                    vT[h] = vT_v[h].astype(jnp.bfloat16)
                qh = qc[:, h * DH:(h + 1) * DH]
                vh = vT[h, pl.ds(0, (c + 1) * CHUNK), :]
                kd = kT[h, pl.ds(c * CHUNK, CHUNK), :]
                sd = lax.dot_general(
                    qh, kd, (((1,), (1,)), ((), ())),
                    preferred_element_type=jnp.float32)
                wd = jnp.exp(sd + diag_bias)
                if c > 0:
                    kf = kT[h, pl.ds(0, c * CHUNK), :]
                    sf = lax.dot_general(
                        qh, kf, (((1,), (1,)), ((), ())),
                        preferred_element_type=jnp.float32)
                    w = jnp.concatenate([jnp.exp(sf), wd], axis=1)
                else:
                    w = wd
                denom = jnp.sum(w, axis=-1, keepdims=True)
                ctx_raw = jax.lax.dot(
                    w.astype(jnp.bfloat16), vh,
                    preferred_element_type=jnp.float32)
                ctx_cols.append(ctx_raw * (1.0 / denom))
            ctx = jnp.concatenate(ctx_cols, axis=1).astype(jnp.bfloat16)
            pc = jax.lax.dot(ctx, wob[...],
                             preferred_element_type=jnp.float32)
            pcb = pc.astype(jnp.bfloat16)

            for sub in range(SPC):
                j = SPC * c + sub
                sub_val = pcb[sub * SUB:(sub + 1) * SUB, :]

                @pl.when(j % N_DEV == my)
                def _():
                    rs_ref[(j % N_DEV) * SPC + j // N_DEV] = sub_val

                @pl.when(j % N_DEV != my)
                def _():
                    stage_ref[j] = sub_val
                    rs_send_desc(j).start()

            if c >= 1:
                for j in range(SPC * (c - 1), SPC * c):
                    @pl.when(j % N_DEV == my)
                    def _():
                        reduce_and_broadcast(j)

        for j in range(N_SUB - SPC, N_SUB):
            @pl.when(j % N_DEV == my)
            def _():
                reduce_and_broadcast(j)

        for j in range(N_SUB):
            @pl.when(j % N_DEV != my)
            def _():
                pltpu.make_async_remote_copy(
                    src_ref=ag_ref.at[j],
                    dst_ref=ag_ref.at[j],
                    send_sem=ag_send_sems.at[j],
                    recv_sem=ag_recv_sems.at[j],
                    device_id=(j % N_DEV,),
                    device_id_type=pl.DeviceIdType.MESH,
                ).wait_recv()
                out_ref[0, pl.ds(j * SUB, SUB), :] = (
                    ag_ref[j].astype(jnp.float32))

            @pl.when(j % N_DEV == my)
            def _():
                out_ref[0, pl.ds(j * SUB, SUB), :] = (
                    ag_ref[j].astype(jnp.float32))

        for j in range(N_SUB):
            @pl.when(j % N_DEV != my)
            def _():
                rs_send_desc(j).wait_send()
        for q in range(SPC):
            for d in range(N_DEV):
                @pl.when(d != my)
                def _():
                    pltpu.make_async_remote_copy(
                        src_ref=ag_ref.at[q],
                        dst_ref=ag_ref.at[q],
                        send_sem=ag_send_sems.at[q * N_DEV + d],
                        recv_sem=ag_recv_sems.at[q],
                        device_id=(d,),
                        device_id_type=pl.DeviceIdType.MESH,
                    ).wait_send()

    return pl.pallas_call(
        body,
        out_shape=jax.ShapeDtypeStruct((1, SQ, D_MODEL), jnp.float32),
        in_specs=[pl.BlockSpec(memory_space=pl.ANY)] * 5,
        out_specs=pl.BlockSpec(memory_space=pltpu.VMEM),
        scratch_shapes=[
            pltpu.VMEM((SQ, D_MODEL), jnp.float32),
            pltpu.VMEM((D_MODEL, D_QKV), jnp.float32),
            pltpu.VMEM((H_LOC, SKV, DH), jnp.float32),
            pltpu.VMEM((H_LOC, SKV, DH), jnp.float32),
            pltpu.VMEM((D_QKV, D_MODEL), jnp.float32),
            pltpu.VMEM((D_MODEL, D_QKV), jnp.bfloat16),
            pltpu.VMEM((D_QKV, D_MODEL), jnp.bfloat16),
            pltpu.VMEM((H_LOC, SKV, DH), jnp.bfloat16),
            pltpu.VMEM((H_LOC, SKV, DH), jnp.bfloat16),
            pltpu.VMEM((N_SUB, SUB, D_MODEL), jnp.bfloat16),
            pltpu.VMEM((N_SUB, SUB, D_MODEL), jnp.bfloat16),
            pltpu.VMEM((N_SUB, SUB, D_MODEL), jnp.bfloat16),
            pltpu.SemaphoreType.DMA((3,)),
            pltpu.SemaphoreType.DMA((2 * H_LOC,)),
            pltpu.SemaphoreType.DMA((N_SUB,)),
            pltpu.SemaphoreType.DMA((N_SUB,)),
            pltpu.SemaphoreType.DMA((N_SUB,)),
            pltpu.SemaphoreType.DMA((N_SUB,)),
        ],
        compiler_params=pltpu.CompilerParams(
            collective_id=0, vmem_limit_bytes=110 * 1024 * 1024),
    )(x, Wq, K_ext, V_ext, Wo)


# device time: 27321 ns/iter; 2.3024x vs baseline; 1.6818x over previous
import jax
import jax.numpy as jnp
from jax import lax
from jax.experimental import pallas as pl
from jax.experimental.pallas import tpu as pltpu

N_DEV = 4
SQ = 1024
SKV = 1024
H_LOC = 8
DH = 128
D_MODEL = 1024
D_QKV = H_LOC * DH
SCALE = 0.08838834764831843
LOG2E = 1.4426950408889634
BLK = 64
CHUNK = 256
SUB = 64
N_SUB = SQ // SUB
SPC = CHUNK // SUB


def kernel(x, Wq, K_ext, V_ext, Wo):
    def body(x_hbm, wq_hbm, k_hbm, v_hbm, wo_hbm, out_ref,
             x_v, wq_v, kT_v, vT_v, wo_v,
             wqb, wob, kT, vT,
             stage_ref, rs_ref,
             load_sems, tr_sems,
             rs_send_sems, rs_recv_sems, ag_send_sems, ag_recv_sems):
        my = lax.axis_index("i")

        tr_k = [pltpu.make_async_copy(
            k_hbm.at[0, :, h, :], kT_v.at[h], tr_sems.at[h])
            for h in range(H_LOC)]
        tr_v = [pltpu.make_async_copy(
            v_hbm.at[0, :, h, :], vT_v.at[h], tr_sems.at[H_LOC + h])
            for h in range(H_LOC)]
        for h in range(H_LOC):
            tr_k[h].start()
            tr_v[h].start()
        cp_x = pltpu.make_async_copy(x_hbm.at[0], x_v, load_sems.at[0])
        cp_wq = pltpu.make_async_copy(
            wq_hbm.at[:, pl.ds(my * D_QKV, D_QKV)], wq_v, load_sems.at[1])
        cp_wo = pltpu.make_async_copy(
            wo_hbm.at[pl.ds(my * D_QKV, D_QKV), :], wo_v, load_sems.at[2])
        for cp in (cp_x, cp_wq, cp_wo):
            cp.start()

        barrier_sem = pltpu.get_barrier_semaphore()
        for d in range(1, N_DEV):
            pl.semaphore_signal(
                barrier_sem, inc=1,
                device_id=(lax.rem(my + d, N_DEV),),
                device_id_type=pl.DeviceIdType.MESH,
            )
        pl.semaphore_wait(barrier_sem, N_DEV - 1)

        cp_wq.wait()
        wqb[...] = (wq_v[...] * (SCALE * LOG2E)).astype(jnp.bfloat16)
        cp_wo.wait()
        wob[...] = wo_v[...].astype(jnp.bfloat16)
        cp_x.wait()

        def rs_send_desc(j):
            return pltpu.make_async_remote_copy(
                src_ref=stage_ref.at[j],
                dst_ref=rs_ref.at[my * SPC + j // N_DEV],
                send_sem=rs_send_sems.at[j],
                recv_sem=rs_recv_sems.at[my * SPC + j // N_DEV],
                device_id=(j % N_DEV,),
                device_id_type=pl.DeviceIdType.MESH,
            )

        def rs_recv_desc(s, q):
            return pltpu.make_async_remote_copy(
                src_ref=stage_ref.at[s],
                dst_ref=rs_ref.at[s * SPC + q],
                send_sem=rs_send_sems.at[s],
                recv_sem=rs_recv_sems.at[s * SPC + q],
                device_id=(s,),
                device_id_type=pl.DeviceIdType.MESH,
            )

        def ag_send_desc(j, d):
            return pltpu.make_async_remote_copy(
                src_ref=out_ref.at[0, pl.ds(j * SUB, SUB), :],
                dst_ref=out_ref.at[0, pl.ds(j * SUB, SUB), :],
                send_sem=ag_send_sems.at[(j // N_DEV) * N_DEV + d],
                recv_sem=ag_recv_sems.at[j],
                device_id=(d,),
                device_id_type=pl.DeviceIdType.MESH,
            )

        def reduce_and_broadcast(j):
            own = j % N_DEV
            q = j // N_DEV
            for s_id in range(N_DEV):
                if s_id != own:
                    rs_recv_desc(s_id, q).wait_recv()
            red = (rs_ref[0 * SPC + q].astype(jnp.float32)
                   + rs_ref[1 * SPC + q].astype(jnp.float32)
                   + rs_ref[2 * SPC + q].astype(jnp.float32)
                   + rs_ref[3 * SPC + q].astype(jnp.float32))
            out_ref[0, pl.ds(j * SUB, SUB), :] = red.astype(jnp.bfloat16)
            for d in range(N_DEV):
                if d != own:
                    ag_send_desc(j, d).start()

        rb = lax.broadcasted_iota(jnp.int32, (CHUNK, CHUNK), 0) // BLK
        cb = lax.broadcasted_iota(jnp.int32, (CHUNK, CHUNK), 1) // BLK
        diag_bias = jnp.where(cb <= rb, 0.0, -1e9).astype(jnp.float32)

        for c in range(N_DEV):
            xc = x_v[pl.ds(c * CHUNK, CHUNK), :].astype(jnp.bfloat16)
            qc = jax.lax.dot(xc, wqb[...],
                             preferred_element_type=jnp.float32
                             ).astype(jnp.bfloat16)

            ctx_cols = []
            for h in range(H_LOC):
                if c == 0:
                    tr_k[h].wait()
                    kT[h] = kT_v[h].astype(jnp.bfloat16)
                    tr_v[h].wait()
                    vT[h] = vT_v[h].astype(jnp.bfloat16)
                qh = qc[:, h * DH:(h + 1) * DH]
                vh = vT[h, pl.ds(0, (c + 1) * CHUNK), :]
                kd = kT[h, pl.ds(c * CHUNK, CHUNK), :]
                sd = lax.dot_general(
                    qh, kd, (((1,), (1,)), ((), ())),
                    preferred_element_type=jnp.float32)
                wd = jnp.exp2(sd + diag_bias)
                if c > 0:
                    kf = kT[h, pl.ds(0, c * CHUNK), :]
                    sf = lax.dot_general(
                        qh, kf, (((1,), (1,)), ((), ())),
                        preferred_element_type=jnp.float32)
                    w = jnp.concatenate([jnp.exp2(sf), wd], axis=1)
                else:
                    w = wd
                denom = jnp.sum(w, axis=-1, keepdims=True)
                ctx_raw = jax.lax.dot(
                    w.astype(jnp.bfloat16), vh,
                    preferred_element_type=jnp.float32)
                ctx_cols.append(ctx_raw * (1.0 / denom))
            ctx = jnp.concatenate(ctx_cols, axis=1).astype(jnp.bfloat16)
            pc = jax.lax.dot(ctx, wob[...],
                             preferred_element_type=jnp.float32)
            pcb = pc.astype(jnp.bfloat16)

            for sub in range(SPC):
                j = SPC * c + sub
                sub_val = pcb[sub * SUB:(sub + 1) * SUB, :]

                @pl.when(j % N_DEV == my)
                def _():
                    rs_ref[(j % N_DEV) * SPC + j // N_DEV] = sub_val

                @pl.when(j % N_DEV != my)
                def _():
                    stage_ref[j] = sub_val
                    rs_send_desc(j).start()

            if c >= 1:
                for j in range(SPC * (c - 1), SPC * c):
                    @pl.when(j % N_DEV == my)
                    def _():
                        reduce_and_broadcast(j)

        for j in range(N_SUB - SPC, N_SUB):
            @pl.when(j % N_DEV == my)
            def _():
                reduce_and_broadcast(j)

        for j in range(N_SUB):
            @pl.when(j % N_DEV != my)
            def _():
                pltpu.make_async_remote_copy(
                    src_ref=out_ref.at[0, pl.ds(j * SUB, SUB), :],
                    dst_ref=out_ref.at[0, pl.ds(j * SUB, SUB), :],
                    send_sem=ag_send_sems.at[j],
                    recv_sem=ag_recv_sems.at[j],
                    device_id=(j % N_DEV,),
                    device_id_type=pl.DeviceIdType.MESH,
                ).wait_recv()

        for j in range(N_SUB):
            @pl.when(j % N_DEV != my)
            def _():
                rs_send_desc(j).wait_send()
        for q in range(SPC):
            for d in range(N_DEV):
                @pl.when(d != my)
                def _():
                    pltpu.make_async_remote_copy(
                        src_ref=out_ref.at[0, pl.ds(q * SUB, SUB), :],
                        dst_ref=out_ref.at[0, pl.ds(q * SUB, SUB), :],
                        send_sem=ag_send_sems.at[q * N_DEV + d],
                        recv_sem=ag_recv_sems.at[q],
                        device_id=(d,),
                        device_id_type=pl.DeviceIdType.MESH,
                    ).wait_send()

    return pl.pallas_call(
        body,
        out_shape=jax.ShapeDtypeStruct((1, SQ, D_MODEL), jnp.bfloat16),
        in_specs=[pl.BlockSpec(memory_space=pl.ANY)] * 5,
        out_specs=pl.BlockSpec(memory_space=pltpu.VMEM),
        scratch_shapes=[
            pltpu.VMEM((SQ, D_MODEL), jnp.float32),
            pltpu.VMEM((D_MODEL, D_QKV), jnp.float32),
            pltpu.VMEM((H_LOC, SKV, DH), jnp.float32),
            pltpu.VMEM((H_LOC, SKV, DH), jnp.float32),
            pltpu.VMEM((D_QKV, D_MODEL), jnp.float32),
            pltpu.VMEM((D_MODEL, D_QKV), jnp.bfloat16),
            pltpu.VMEM((D_QKV, D_MODEL), jnp.bfloat16),
            pltpu.VMEM((H_LOC, SKV, DH), jnp.bfloat16),
            pltpu.VMEM((H_LOC, SKV, DH), jnp.bfloat16),
            pltpu.VMEM((N_SUB, SUB, D_MODEL), jnp.bfloat16),
            pltpu.VMEM((N_SUB, SUB, D_MODEL), jnp.bfloat16),
            pltpu.SemaphoreType.DMA((3,)),
            pltpu.SemaphoreType.DMA((2 * H_LOC,)),
            pltpu.SemaphoreType.DMA((N_SUB,)),
            pltpu.SemaphoreType.DMA((N_SUB,)),
            pltpu.SemaphoreType.DMA((N_SUB,)),
            pltpu.SemaphoreType.DMA((N_SUB,)),
        ],
        compiler_params=pltpu.CompilerParams(
            collective_id=0, vmem_limit_bytes=110 * 1024 * 1024),
    )(x, Wq, K_ext, V_ext, Wo)
